# baseline (speedup 1.0000x reference)
import functools
import math

import jax
import jax.numpy as jnp
from jax import lax
from jax.experimental import pallas as pl
from jax.experimental.pallas import tpu as pltpu
from jax.experimental.pallas import tpu_sc as plsc

N = 50000
E = 800000
D = 64
ATOM_VOCAB = 100
BOND_VOCAB = 16
G = 2000
LOG2 = math.log(2.0)
NEG_BIG = -3.0e38
E_PAD = 819200
N_PAD = 51200
NQ_PAD = N_PAD
DH = 32
READOUT_ = 64


def _softplus(v):
    return jnp.maximum(v, 0.0) + jnp.log1p(jnp.exp(-jnp.abs(v)))


_EMB_BLK = 2000


def _emb_body(x_ref, ea_ref, aw_ref, bw_ref, x_out, b_out):
    xv = x_ref[0, 0, :]
    av = ea_ref[0, 0, :]
    oh_a = (xv[:, None] == jax.lax.broadcasted_iota(jnp.int32, (1, ATOM_VOCAB), 1)
            ).astype(jnp.float32)
    oh_b = (av[:, None] == jax.lax.broadcasted_iota(jnp.int32, (1, BOND_VOCAB), 1)
            ).astype(jnp.float32)
    x_out[...] = _softplus(jnp.dot(oh_a, aw_ref[...],
                                   preferred_element_type=jnp.float32, precision=jax.lax.Precision.HIGHEST))
    b_out[...] = _softplus(jnp.dot(oh_b, bw_ref[...],
                                   preferred_element_type=jnp.float32, precision=jax.lax.Precision.HIGHEST))


def _embeddings(x, ea_first, emb_atom_W, emb_bond_W):
    nb = N // _EMB_BLK
    x3 = x.reshape(nb, 1, _EMB_BLK)
    e3 = ea_first.reshape(nb, 1, _EMB_BLK)
    return pl.pallas_call(
        _emb_body,
        grid=(nb,),
        in_specs=[
            pl.BlockSpec((1, 1, _EMB_BLK), lambda i: (i, 0, 0)),
            pl.BlockSpec((1, 1, _EMB_BLK), lambda i: (i, 0, 0)),
            pl.BlockSpec((ATOM_VOCAB, D), lambda i: (0, 0)),
            pl.BlockSpec((BOND_VOCAB, D), lambda i: (0, 0)),
        ],
        out_specs=[
            pl.BlockSpec((_EMB_BLK, D), lambda i: (i, 0)),
            pl.BlockSpec((_EMB_BLK, D), lambda i: (i, 0)),
        ],
        out_shape=[
            jax.ShapeDtypeStruct((N, D), jnp.float32),
            jax.ShapeDtypeStruct((N, D), jnp.float32),
        ],
    )(x3, e3, emb_atom_W, emb_bond_W)


_ROW_BLK = 5000


def _dense3_body(b_ref, x_ref, w1_ref, wb_ref, wm_ref, bb_ref, k_ref, a_out, c_out, p_out):
    b = b_ref[...]
    x = x_ref[...]
    a_out[...] = jnp.dot(b, w1_ref[...], preferred_element_type=jnp.float32, precision=jax.lax.Precision.HIGHEST) + bb_ref[...]
    c_out[...] = jnp.dot(b, wb_ref[...], preferred_element_type=jnp.float32, precision=jax.lax.Precision.HIGHEST)
    p_out[...] = jnp.dot(x, wm_ref[...], preferred_element_type=jnp.float32) + k_ref[...]


def _dense3(B, X, W1, Wb, Wm, b_bond, kvec):
    nb = N // _ROW_BLK
    return pl.pallas_call(
        _dense3_body,
        grid=(nb,),
        in_specs=[
            pl.BlockSpec((_ROW_BLK, D), lambda i: (i, 0)),
            pl.BlockSpec((_ROW_BLK, D), lambda i: (i, 0)),
            pl.BlockSpec((D, D), lambda i: (0, 0)),
            pl.BlockSpec((D, D), lambda i: (0, 0)),
            pl.BlockSpec((D, D), lambda i: (0, 0)),
            pl.BlockSpec((1, D), lambda i: (0, 0)),
            pl.BlockSpec((1, D), lambda i: (0, 0)),
        ],
        out_specs=[
            pl.BlockSpec((_ROW_BLK, D), lambda i: (i, 0)),
            pl.BlockSpec((_ROW_BLK, D), lambda i: (i, 0)),
            pl.BlockSpec((_ROW_BLK, D), lambda i: (i, 0)),
        ],
        out_shape=[
            jax.ShapeDtypeStruct((N, D), jnp.float32),
            jax.ShapeDtypeStruct((N, D), jnp.float32),
            jax.ShapeDtypeStruct((N, D), jnp.float32),
        ],
    )(B, X, W1, Wb, Wm, b_bond, kvec)


def _postmax_body(a_ref, m_ref, we_ref, qc_ref, bn_out, qp_out):
    m = m_ref[...]
    bn = jnp.where(m < -1.0e37, jnp.float32(LOG2), _softplus(a_ref[...] + m))
    bn_out[...] = bn
    qp_out[...] = (jnp.dot(bn, we_ref[...], preferred_element_type=jnp.float32)
                   - qc_ref[...])


def _postmax(A, M, W_edge, qc):
    nb = N // _ROW_BLK
    return pl.pallas_call(
        _postmax_body,
        grid=(nb,),
        in_specs=[
            pl.BlockSpec((_ROW_BLK, D), lambda i: (i, 0)),
            pl.BlockSpec((_ROW_BLK, D), lambda i: (i, 0)),
            pl.BlockSpec((D, D), lambda i: (0, 0)),
            pl.BlockSpec((1, D), lambda i: (0, 0)),
        ],
        out_specs=[
            pl.BlockSpec((_ROW_BLK, D), lambda i: (i, 0)),
            pl.BlockSpec((_ROW_BLK, D), lambda i: (i, 0)),
        ],
        out_shape=[
            jax.ShapeDtypeStruct((N, D), jnp.float32),
            jax.ShapeDtypeStruct((N, D), jnp.float32),
        ],
    )(A, M, W_edge, qc)


def _postsum_body(s0_ref, s1_ref, s2_ref, s3_ref, x_ref, k_ref, wm_ref, xn_out, pn_out):
    sfull = jnp.concatenate([s0_ref[0], s1_ref[0], s2_ref[0], s3_ref[0]], axis=-1)
    xn = _softplus(sfull + x_ref[...])
    xn_out[...] = xn
    pn_out[...] = jnp.dot(xn, wm_ref[...], preferred_element_type=jnp.float32) + k_ref[...]


def _postsum(S_a, S_b, X, Kvec, Wm):
    nb = N // _ROW_BLK
    return pl.pallas_call(
        _postsum_body,
        grid=(nb,),
        in_specs=[
            pl.BlockSpec((1, _ROW_BLK, 16), lambda i: (0, i, 0)),
            pl.BlockSpec((1, _ROW_BLK, 16), lambda i: (1, i, 0)),
            pl.BlockSpec((1, _ROW_BLK, 16), lambda i: (0, i, 0)),
            pl.BlockSpec((1, _ROW_BLK, 16), lambda i: (1, i, 0)),
            pl.BlockSpec((_ROW_BLK, D), lambda i: (i, 0)),
            pl.BlockSpec((1, D), lambda i: (0, 0)),
            pl.BlockSpec((D, D), lambda i: (0, 0)),
        ],
        out_specs=[
            pl.BlockSpec((_ROW_BLK, D), lambda i: (i, 0)),
            pl.BlockSpec((_ROW_BLK, D), lambda i: (i, 0)),
        ],
        out_shape=[
            jax.ShapeDtypeStruct((N, D), jnp.float32),
            jax.ShapeDtypeStruct((N, D), jnp.float32),
        ],
    )(S_a, S_a, S_b, S_b, X, Kvec, Wm)


def _readout_body(p0_ref, p1_ref, w0_ref, b0_ref, w1_ref, b1_ref, wo_ref, bo_ref, o_ref):
    pooled = jnp.concatenate([p0_ref[0], p1_ref[0]], axis=-1)
    h = _softplus(jnp.dot(pooled, w0_ref[...],
                          preferred_element_type=jnp.float32) + b0_ref[...])
    h = _softplus(jnp.dot(h, w1_ref[...],
                          preferred_element_type=jnp.float32) + b1_ref[...])
    o_ref[...] = jnp.dot(h, wo_ref[...],
                         preferred_element_type=jnp.float32) + bo_ref[...]


def _readout(pooled_pl, W_r0, b_r0, W_r1, b_r1, W_out, b_out):
    return pl.pallas_call(
        _readout_body,
        grid=(1,),
        in_specs=[
            pl.BlockSpec((1, G, DH), lambda i: (0, 0, 0)),
            pl.BlockSpec((1, G, DH), lambda i: (1, 0, 0)),
            pl.BlockSpec((D, READOUT_), lambda i: (0, 0)),
            pl.BlockSpec((1, READOUT_), lambda i: (0, 0)),
            pl.BlockSpec((READOUT_, READOUT_), lambda i: (0, 0)),
            pl.BlockSpec((1, READOUT_), lambda i: (0, 0)),
            pl.BlockSpec((READOUT_, 8), lambda i: (0, 0)),
            pl.BlockSpec((1, 8), lambda i: (0, 0)),
        ],
        out_specs=pl.BlockSpec((G, 8), lambda i: (0, 0)),
        out_shape=jax.ShapeDtypeStruct((G, 8), jnp.float32),
    )(pooled_pl, pooled_pl, W_r0, b_r0.reshape(1, -1), W_r1, b_r1.reshape(1, -1),
      jnp.pad(W_out, ((0, 0), (0, 7))), jnp.pad(b_out, (0, 7)).reshape(1, -1))


_FILL_BLK = 25000


def _fill_body(o_ref):
    o_ref[...] = jnp.full((_FILL_BLK, D), jnp.float32(LOG2))


def _fill_tail():
    nb = (E - N) // _FILL_BLK
    return pl.pallas_call(
        _fill_body,
        grid=(nb,),
        out_specs=pl.BlockSpec((_FILL_BLK, D), lambda i: (i, 0)),
        out_shape=jax.ShapeDtypeStruct((E - N, D), jnp.float32),
    )()


_SUM_CHUNKS = E_PAD // 16 // 1024
_Q_CHUNKS = NQ_PAD // 16 // 128


def _sc_sum_body(half, srcp2, dstp2, p_pl, q_pl, dsth2, s_out,
                 sidx, didx, qdid, rows, qrows, zbuf, acc, gsem, ssem):
    c = lax.axis_index("c")
    s = lax.axis_index("s")

    def zb(i, carry):
        zbuf[i, pl.ds(0, 16)] = jnp.zeros((16,), jnp.float32)
        return carry

    lax.fori_loop(0, 800, zb, 0)
    for k in range(4):
        pltpu.sync_copy(zbuf, acc.at[pl.ds(s * 3200 + k * 800, 800)])
    plsc.subcore_barrier()

    p_c = p_pl.at[2 * half + c]
    q_c = q_pl.at[2 * half + c]
    base_row = s * (_SUM_CHUNKS * 8)

    def chunk(ci, carry):
        r0 = base_row + ci * 8
        pltpu.sync_copy(srcp2.at[pl.ds(r0, 8)], sidx)
        pltpu.sync_copy(dstp2.at[pl.ds(r0, 8)], didx)
        gets = [pltpu.async_copy(p_c.at[sidx.at[j]], rows.at[j], gsem)
                for j in range(8)]
        for cp in gets:
            cp.wait()
        puts = [pltpu.async_copy(rows.at[j], acc.at[didx.at[j]], ssem, add=True)
                for j in range(8)]
        for cp in puts:
            cp.wait()

        @pl.when(ci < _Q_CHUNKS)
        def _():
            pltpu.sync_copy(dsth2.at[pl.ds(s * _Q_CHUNKS + ci, 1)], qdid)
            pltpu.sync_copy(q_c.at[pl.ds(s * 3200 + ci * 128, 128)], qrows)
            pltpu.sync_copy(qrows, acc.at[qdid.at[0]], add=True)

        return carry

    lax.fori_loop(0, _SUM_CHUNKS, chunk, 0)
    plsc.subcore_barrier()
    pltpu.sync_copy(acc.at[pl.ds(s * 3200, 3200)],
                    s_out.at[c].at[pl.ds(s * 3200, 3200)])


def _sc_sum(half, srcp2, dstp2, p_pl, q_pl, dsth2):
    f = pl.kernel(
        functools.partial(_sc_sum_body, half),
        out_type=jax.ShapeDtypeStruct((2, N_PAD, 16), jnp.float32),
        mesh=plsc.VectorSubcoreMesh(core_axis_name="c", subcore_axis_name="s"),
        compiler_params=pltpu.CompilerParams(use_tc_tiling_on_sc=False),
        scratch_types=[
            pltpu.VMEM((8, 128), jnp.int32),
            pltpu.VMEM((8, 128), jnp.int32),
            pltpu.VMEM((1, 128), jnp.int32),
            pltpu.VMEM((8, 128, 16), jnp.float32),
            pltpu.VMEM((128, 16), jnp.float32),
            pltpu.VMEM((800, 16), jnp.float32),
            pltpu.VMEM_SHARED((N_PAD, 16), jnp.float32),
            pltpu.SemaphoreType.DMA,
            pltpu.SemaphoreType.DMA,
        ],
    )
    return f(srcp2, dstp2, p_pl, q_pl, dsth2)


def _planes(a):
    return jnp.stack([a[:, :DH], a[:, DH:]])


def _planes4(a):
    return jnp.stack([a[:, 0:16], a[:, 16:32], a[:, 32:48], a[:, 48:64]])


_GACC = 2048


def _sc_pool_body(x_pl, batch2, p_out, bidx, xrows, zbuf, acc, ssem):
    c = lax.axis_index("c")
    s = lax.axis_index("s")

    def zb(i, carry):
        zbuf[i, pl.ds(0, 16)] = jnp.zeros((16,), jnp.float32)
        zbuf[i, pl.ds(16, 16)] = jnp.zeros((16,), jnp.float32)
        return carry

    lax.fori_loop(0, 128, zb, 0)
    pltpu.sync_copy(zbuf, acc.at[pl.ds(s * 128, 128)])
    plsc.subcore_barrier()
    x_c = x_pl.at[c]

    def chunk(ci, carry):
        pltpu.sync_copy(batch2.at[pl.ds(s * 25 + ci, 1)], bidx)
        pltpu.sync_copy(x_c.at[pl.ds(s * 3200 + ci * 128, 128)], xrows)
        pltpu.sync_copy(xrows, acc.at[bidx.at[0]], add=True)
        return carry

    lax.fori_loop(0, 25, chunk, 0)
    plsc.subcore_barrier()
    pltpu.sync_copy(acc.at[pl.ds(s * 128, 128)],
                    p_out.at[c].at[pl.ds(s * 128, 128)])


def _sc_pool(x_pl, batch2):
    f = pl.kernel(
        _sc_pool_body,
        out_type=jax.ShapeDtypeStruct((2, _GACC, DH), jnp.float32),
        mesh=plsc.VectorSubcoreMesh(core_axis_name="c", subcore_axis_name="s"),
        compiler_params=pltpu.CompilerParams(use_tc_tiling_on_sc=False),
        scratch_types=[
            pltpu.VMEM((1, 128), jnp.int32),
            pltpu.VMEM((128, DH), jnp.float32),
            pltpu.VMEM((128, DH), jnp.float32),
            pltpu.VMEM_SHARED((_GACC, DH), jnp.float32),
            pltpu.SemaphoreType.DMA,
        ],
    )
    return f(x_pl, batch2)


_MROWS = 1600
_MCH = 16
_MCHUNKS = E_PAD // (_MCH * 128)
_PEND = 2560


def _sc_max_body(srcp2, dstp2, c_hbm, m_out,
                 sidx, didx, pend_loc, pend_src, rows, accf, gsem):
    c = lax.axis_index("c")
    s = lax.axis_index("s")
    w = s * 2 + c
    base = w * _MROWS
    lanes = lax.broadcasted_iota(jnp.int32, (16,), 0)

    def zi(i, carry):
        accf[pl.ds(i * 16, 16)] = jnp.full((16,), NEG_BIG, jnp.float32)
        return carry

    lax.fori_loop(0, (_MROWS + 64) * D // 16, zi, 0)

    def flush(blk, off0):
        b0 = blk * 128
        pltpu.async_copy(c_hbm.at[pend_src.at[pl.ds(b0, 128)]], rows,
                         gsem).wait()
        for g in range(8):
            loc16 = pend_loc[pl.ds(b0 + g * 16, 16)]
            for l in range(16):
                loc = jnp.sum(jnp.where(lanes == l, loc16, 0))
                ab = loc * D
                for k in range(4):
                    cur = accf[pl.ds(ab + k * 16, 16)]
                    val = rows[g * 16 + l, pl.ds(k * 16, 16)]
                    accf[pl.ds(ab + k * 16, 16)] = jnp.maximum(cur, val)
        return off0

    def chunk(ci, off):
        r0 = ci * _MCH
        pltpu.sync_copy(srcp2.at[pl.ds(r0, _MCH)], sidx)
        pltpu.sync_copy(dstp2.at[pl.ds(r0, _MCH)], didx)
        for j in range(_MCH):
            for v in range(8):
                dv = didx[j, pl.ds(v * 16, 16)]
                sv = sidx[j, pl.ds(v * 16, 16)]
                loc = dv - base
                m = plsc.bitcast(loc, jnp.uint32) < jnp.uint32(_MROWS)
                cnt = jnp.sum(m.astype(jnp.int32))

                @pl.when(cnt > 0)
                def _(off=off, loc=loc, sv=sv, m=m):
                    key = jnp.where(m, loc, jnp.int32(0x7FFFFFF0))
                    ks, vs = plsc.sort_key_val(key, sv)
                    pend_loc[pl.ds(off, 16)] = ks
                    pend_src[pl.ds(off, 16)] = vs

                off = off + cnt
        nblk = off >> 7
        lax.fori_loop(0, nblk, flush, off)
        rem = off & 127

        def mv(l, carry):
            pend_loc[pl.ds(l * 16, 16)] = pend_loc[pl.ds(nblk * 128 + l * 16, 16)]
            pend_src[pl.ds(l * 16, 16)] = pend_src[pl.ds(nblk * 128 + l * 16, 16)]
            return carry

        lax.fori_loop(0, 8, mv, 0)
        return rem

    off = lax.fori_loop(0, _MCHUNKS, chunk, jnp.int32(0))

    def padk(k, carry):
        pend_loc[pl.ds(off + k * 16, 16)] = jnp.full((16,), _MROWS, jnp.int32)
        pend_src[pl.ds(off + k * 16, 16)] = (lanes + 16 * k) * 32
        return carry

    lax.fori_loop(0, 8, padk, 0)
    lax.fori_loop(0, (off + 127) >> 7, flush, 0)

    pltpu.sync_copy(accf.at[pl.ds(0, _MROWS * D)],
                    m_out.at[pl.ds(base * D, _MROWS * D)])


def _sc_max(srcp2, dstp2, C):
    f = pl.kernel(
        _sc_max_body,
        out_type=jax.ShapeDtypeStruct((N_PAD * D,), jnp.float32),
        mesh=plsc.VectorSubcoreMesh(core_axis_name="c", subcore_axis_name="s"),
        compiler_params=pltpu.CompilerParams(use_tc_tiling_on_sc=False,
                                             needs_layout_passes=False),
        scratch_types=[
            pltpu.VMEM((_MCH, 128), jnp.int32),
            pltpu.VMEM((_MCH, 128), jnp.int32),
            pltpu.VMEM((_PEND,), jnp.int32),
            pltpu.VMEM((_PEND,), jnp.int32),
            pltpu.VMEM((128, D), jnp.float32),
            pltpu.VMEM(((_MROWS + 64) * D,), jnp.float32),
            pltpu.SemaphoreType.DMA,
        ],
    )
    return f(srcp2, dstp2, C)


def kernel(x, edge_attr, edge_index, batch, emb_atom_W, emb_bond_W, W_bond, b_bond,
           W_msg, b_msg, W_edge, b_edge, W_r0, b_r0, W_r1, b_r1, W_out, b_out):
    src = edge_index[0]
    dst = edge_index[1]

    W1 = W_bond[:D] - W_bond[D:]
    Wb = W_bond[D:]
    cpad = _softplus(jnp.zeros((256, D), jnp.float32))
    qc = jnp.dot(cpad, W_edge)[:1]
    Kvec = (b_msg + b_edge).reshape(1, D) + qc
    bb = b_bond.reshape(1, D)

    X, B = _embeddings(x, edge_attr[:N], emb_atom_W, emb_bond_W)

    pad_e = E_PAD - E
    spread_e = N + (jnp.arange(pad_e, dtype=jnp.int32) % 1024)
    srcp2 = jnp.concatenate([src, jnp.arange(pad_e, dtype=jnp.int32) % 1024]
                            ).reshape(E_PAD // 128, 128)
    dstp2 = jnp.concatenate([dst, spread_e]).reshape(E_PAD // 128, 128)
    spread_q = N + (jnp.arange(NQ_PAD - N, dtype=jnp.int32) % 1024)
    dsth2 = jnp.concatenate([dst[:N], spread_q]).reshape(NQ_PAD // 128, 128)

    for _ in range(2):
        A, C, P = _dense3(B, X, W1, Wb, W_msg, bb, Kvec)
        M2 = _sc_max(srcp2, dstp2, C).reshape(N_PAD, D)
        B, Qp = _postmax(A, M2, W_edge, qc)
        q_pl = _planes4(jnp.pad(Qp, ((0, NQ_PAD - N), (0, 0))))
        p_pl = _planes4(P)
        S_a = _sc_sum(0, srcp2, dstp2, p_pl, q_pl, dsth2)
        S_b = _sc_sum(1, srcp2, dstp2, p_pl, q_pl, dsth2)
        X, P = _postsum(S_a, S_b, X, Kvec, W_msg)

    out_atom = X
    out_bond = jnp.concatenate([B, _fill_tail()], axis=0)

    x_pl = _planes(jnp.pad(out_atom, ((0, N_PAD - N), (0, 0))))
    spread_b = G + (jnp.arange(N_PAD - N, dtype=jnp.int32) % 48)
    batch2 = jnp.concatenate([batch.astype(jnp.int32), spread_b]
                             ).reshape(N_PAD // 128, 128)
    pooled_pl = _sc_pool(x_pl, batch2)
    out = _readout(pooled_pl, W_r0, b_r0, W_r1, b_r1, W_out, b_out)[:, :1]
    return (out, out_atom, out_bond)


# speedup vs baseline: 1.3210x; 1.3210x over previous
import functools
import math

import jax
import jax.numpy as jnp
from jax import lax
from jax.experimental import pallas as pl
from jax.experimental.pallas import tpu as pltpu
from jax.experimental.pallas import tpu_sc as plsc

N = 50000
E = 800000
D = 64
ATOM_VOCAB = 100
BOND_VOCAB = 16
G = 2000
LOG2 = math.log(2.0)
NEG_BIG = -3.0e38
E_PAD = 819200
N_PAD = 51200
NQ_PAD = N_PAD
DH = 32
READOUT_ = 64


def _softplus(v):
    return jnp.maximum(v, 0.0) + jnp.log1p(jnp.exp(-jnp.abs(v)))


_EMB_BLK = 2000


def _emb_body(x_ref, ea_ref, aw_ref, bw_ref, x_out, b_out):
    xv = x_ref[0, 0, :]
    av = ea_ref[0, 0, :]
    oh_a = (xv[:, None] == jax.lax.broadcasted_iota(jnp.int32, (1, ATOM_VOCAB), 1)
            ).astype(jnp.float32)
    oh_b = (av[:, None] == jax.lax.broadcasted_iota(jnp.int32, (1, BOND_VOCAB), 1)
            ).astype(jnp.float32)
    x_out[...] = _softplus(jnp.dot(oh_a, aw_ref[...],
                                   preferred_element_type=jnp.float32, precision=jax.lax.Precision.HIGHEST))
    b_out[...] = _softplus(jnp.dot(oh_b, bw_ref[...],
                                   preferred_element_type=jnp.float32, precision=jax.lax.Precision.HIGHEST))


def _embeddings(x, ea_first, emb_atom_W, emb_bond_W):
    nb = N // _EMB_BLK
    x3 = x.reshape(nb, 1, _EMB_BLK)
    e3 = ea_first.reshape(nb, 1, _EMB_BLK)
    return pl.pallas_call(
        _emb_body,
        grid=(nb,),
        in_specs=[
            pl.BlockSpec((1, 1, _EMB_BLK), lambda i: (i, 0, 0)),
            pl.BlockSpec((1, 1, _EMB_BLK), lambda i: (i, 0, 0)),
            pl.BlockSpec((ATOM_VOCAB, D), lambda i: (0, 0)),
            pl.BlockSpec((BOND_VOCAB, D), lambda i: (0, 0)),
        ],
        out_specs=[
            pl.BlockSpec((_EMB_BLK, D), lambda i: (i, 0)),
            pl.BlockSpec((_EMB_BLK, D), lambda i: (i, 0)),
        ],
        out_shape=[
            jax.ShapeDtypeStruct((N, D), jnp.float32),
            jax.ShapeDtypeStruct((N, D), jnp.float32),
        ],
    )(x3, e3, emb_atom_W, emb_bond_W)


_ROW_BLK = 5000


def _dense3_body(b_ref, x_ref, w1_ref, wb_ref, wm_ref, bb_ref, k_ref, a_out, c_out, p_out):
    b = b_ref[...]
    x = x_ref[...]
    a_out[...] = jnp.dot(b, w1_ref[...], preferred_element_type=jnp.float32, precision=jax.lax.Precision.HIGHEST) + bb_ref[...]
    c_out[...] = jnp.dot(b, wb_ref[...], preferred_element_type=jnp.float32, precision=jax.lax.Precision.HIGHEST)
    p_out[...] = jnp.dot(x, wm_ref[...], preferred_element_type=jnp.float32) + k_ref[...]


def _dense3(B, X, W1, Wb, Wm, b_bond, kvec):
    nb = N // _ROW_BLK
    return pl.pallas_call(
        _dense3_body,
        grid=(nb,),
        in_specs=[
            pl.BlockSpec((_ROW_BLK, D), lambda i: (i, 0)),
            pl.BlockSpec((_ROW_BLK, D), lambda i: (i, 0)),
            pl.BlockSpec((D, D), lambda i: (0, 0)),
            pl.BlockSpec((D, D), lambda i: (0, 0)),
            pl.BlockSpec((D, D), lambda i: (0, 0)),
            pl.BlockSpec((1, D), lambda i: (0, 0)),
            pl.BlockSpec((1, D), lambda i: (0, 0)),
        ],
        out_specs=[
            pl.BlockSpec((_ROW_BLK, D), lambda i: (i, 0)),
            pl.BlockSpec((_ROW_BLK, D), lambda i: (i, 0)),
            pl.BlockSpec((_ROW_BLK, D), lambda i: (i, 0)),
        ],
        out_shape=[
            jax.ShapeDtypeStruct((N, D), jnp.float32),
            jax.ShapeDtypeStruct((N, D), jnp.float32),
            jax.ShapeDtypeStruct((N, D), jnp.float32),
        ],
    )(B, X, W1, Wb, Wm, b_bond, kvec)


def _postmax_body(a_ref, m_ref, we_ref, qc_ref, bn_out, qp_out):
    m = m_ref[...]
    bn = jnp.where(m < -1.0e37, jnp.float32(LOG2), _softplus(a_ref[...] + m))
    bn_out[...] = bn
    qp_out[...] = (jnp.dot(bn, we_ref[...], preferred_element_type=jnp.float32)
                   - qc_ref[...])


def _postmax(A, M, W_edge, qc):
    nb = N // _ROW_BLK
    return pl.pallas_call(
        _postmax_body,
        grid=(nb,),
        in_specs=[
            pl.BlockSpec((_ROW_BLK, D), lambda i: (i, 0)),
            pl.BlockSpec((_ROW_BLK, D), lambda i: (i, 0)),
            pl.BlockSpec((D, D), lambda i: (0, 0)),
            pl.BlockSpec((1, D), lambda i: (0, 0)),
        ],
        out_specs=[
            pl.BlockSpec((_ROW_BLK, D), lambda i: (i, 0)),
            pl.BlockSpec((_ROW_BLK, D), lambda i: (i, 0)),
        ],
        out_shape=[
            jax.ShapeDtypeStruct((N, D), jnp.float32),
            jax.ShapeDtypeStruct((N, D), jnp.float32),
        ],
    )(A, M, W_edge, qc)


def _postsum_body(s0_ref, s1_ref, s2_ref, s3_ref, x_ref, k_ref, wm_ref, xn_out, pn_out):
    sfull = jnp.concatenate([s0_ref[0], s1_ref[0], s2_ref[0], s3_ref[0]], axis=-1)
    xn = _softplus(sfull + x_ref[...])
    xn_out[...] = xn
    pn_out[...] = jnp.dot(xn, wm_ref[...], preferred_element_type=jnp.float32) + k_ref[...]


def _postsum(S_a, S_b, X, Kvec, Wm):
    nb = N // _ROW_BLK
    return pl.pallas_call(
        _postsum_body,
        grid=(nb,),
        in_specs=[
            pl.BlockSpec((1, _ROW_BLK, 16), lambda i: (0, i, 0)),
            pl.BlockSpec((1, _ROW_BLK, 16), lambda i: (1, i, 0)),
            pl.BlockSpec((1, _ROW_BLK, 16), lambda i: (0, i, 0)),
            pl.BlockSpec((1, _ROW_BLK, 16), lambda i: (1, i, 0)),
            pl.BlockSpec((_ROW_BLK, D), lambda i: (i, 0)),
            pl.BlockSpec((1, D), lambda i: (0, 0)),
            pl.BlockSpec((D, D), lambda i: (0, 0)),
        ],
        out_specs=[
            pl.BlockSpec((_ROW_BLK, D), lambda i: (i, 0)),
            pl.BlockSpec((_ROW_BLK, D), lambda i: (i, 0)),
        ],
        out_shape=[
            jax.ShapeDtypeStruct((N, D), jnp.float32),
            jax.ShapeDtypeStruct((N, D), jnp.float32),
        ],
    )(S_a, S_a, S_b, S_b, X, Kvec, Wm)


def _readout_body(p0_ref, p1_ref, w0_ref, b0_ref, w1_ref, b1_ref, wo_ref, bo_ref, o_ref):
    pooled = jnp.concatenate([p0_ref[0], p1_ref[0]], axis=-1)
    h = _softplus(jnp.dot(pooled, w0_ref[...],
                          preferred_element_type=jnp.float32) + b0_ref[...])
    h = _softplus(jnp.dot(h, w1_ref[...],
                          preferred_element_type=jnp.float32) + b1_ref[...])
    o_ref[...] = jnp.dot(h, wo_ref[...],
                         preferred_element_type=jnp.float32) + bo_ref[...]


def _readout(pooled_pl, W_r0, b_r0, W_r1, b_r1, W_out, b_out):
    return pl.pallas_call(
        _readout_body,
        grid=(1,),
        in_specs=[
            pl.BlockSpec((1, G, DH), lambda i: (0, 0, 0)),
            pl.BlockSpec((1, G, DH), lambda i: (1, 0, 0)),
            pl.BlockSpec((D, READOUT_), lambda i: (0, 0)),
            pl.BlockSpec((1, READOUT_), lambda i: (0, 0)),
            pl.BlockSpec((READOUT_, READOUT_), lambda i: (0, 0)),
            pl.BlockSpec((1, READOUT_), lambda i: (0, 0)),
            pl.BlockSpec((READOUT_, 8), lambda i: (0, 0)),
            pl.BlockSpec((1, 8), lambda i: (0, 0)),
        ],
        out_specs=pl.BlockSpec((G, 8), lambda i: (0, 0)),
        out_shape=jax.ShapeDtypeStruct((G, 8), jnp.float32),
    )(pooled_pl, pooled_pl, W_r0, b_r0.reshape(1, -1), W_r1, b_r1.reshape(1, -1),
      jnp.pad(W_out, ((0, 0), (0, 7))), jnp.pad(b_out, (0, 7)).reshape(1, -1))


_FILL_BLK = 25000


def _fill_body(o_ref):
    o_ref[...] = jnp.full((_FILL_BLK, D), jnp.float32(LOG2))


def _fill_tail():
    nb = (E - N) // _FILL_BLK
    return pl.pallas_call(
        _fill_body,
        grid=(nb,),
        out_specs=pl.BlockSpec((_FILL_BLK, D), lambda i: (i, 0)),
        out_shape=jax.ShapeDtypeStruct((E - N, D), jnp.float32),
    )()


_SUM_CHUNKS = E_PAD // 16 // 1024
_Q_CHUNKS = NQ_PAD // 16 // 128


def _sc_sum_body(half, srcp2, dstp2, p_pl, q_pl, dsth2, s_out,
                 sidx, didx, qdid, rows, qrows, zbuf, acc, gsem, ssem):
    c = lax.axis_index("c")
    s = lax.axis_index("s")

    def zb(i, carry):
        zbuf[i, pl.ds(0, 16)] = jnp.zeros((16,), jnp.float32)
        return carry

    lax.fori_loop(0, 800, zb, 0)
    for k in range(4):
        pltpu.sync_copy(zbuf, acc.at[pl.ds(s * 3200 + k * 800, 800)])
    plsc.subcore_barrier()

    p_c = p_pl.at[2 * half + c]
    q_c = q_pl.at[2 * half + c]
    base_row = s * (_SUM_CHUNKS * 8)

    def chunk(ci, carry):
        r0 = base_row + ci * 8
        pltpu.sync_copy(srcp2.at[pl.ds(r0, 8)], sidx)
        pltpu.sync_copy(dstp2.at[pl.ds(r0, 8)], didx)
        gets = [pltpu.async_copy(p_c.at[sidx.at[j]], rows.at[j], gsem)
                for j in range(8)]
        for cp in gets:
            cp.wait()
        puts = [pltpu.async_copy(rows.at[j], acc.at[didx.at[j]], ssem, add=True)
                for j in range(8)]
        for cp in puts:
            cp.wait()

        @pl.when(ci < _Q_CHUNKS)
        def _():
            pltpu.sync_copy(dsth2.at[pl.ds(s * _Q_CHUNKS + ci, 1)], qdid)
            pltpu.sync_copy(q_c.at[pl.ds(s * 3200 + ci * 128, 128)], qrows)
            pltpu.sync_copy(qrows, acc.at[qdid.at[0]], add=True)

        return carry

    lax.fori_loop(0, _SUM_CHUNKS, chunk, 0)
    plsc.subcore_barrier()
    pltpu.sync_copy(acc.at[pl.ds(s * 3200, 3200)],
                    s_out.at[c].at[pl.ds(s * 3200, 3200)])


def _sc_sum(half, srcp2, dstp2, p_pl, q_pl, dsth2):
    f = pl.kernel(
        functools.partial(_sc_sum_body, half),
        out_type=jax.ShapeDtypeStruct((2, N_PAD, 16), jnp.float32),
        mesh=plsc.VectorSubcoreMesh(core_axis_name="c", subcore_axis_name="s"),
        compiler_params=pltpu.CompilerParams(use_tc_tiling_on_sc=False),
        scratch_types=[
            pltpu.VMEM((8, 128), jnp.int32),
            pltpu.VMEM((8, 128), jnp.int32),
            pltpu.VMEM((1, 128), jnp.int32),
            pltpu.VMEM((8, 128, 16), jnp.float32),
            pltpu.VMEM((128, 16), jnp.float32),
            pltpu.VMEM((800, 16), jnp.float32),
            pltpu.VMEM_SHARED((N_PAD, 16), jnp.float32),
            pltpu.SemaphoreType.DMA,
            pltpu.SemaphoreType.DMA,
        ],
    )
    return f(srcp2, dstp2, p_pl, q_pl, dsth2)


def _planes(a):
    return jnp.stack([a[:, :DH], a[:, DH:]])


def _planes4(a):
    return jnp.stack([a[:, 0:16], a[:, 16:32], a[:, 32:48], a[:, 48:64]])


_GACC = 2048


def _sc_pool_body(x_pl, batch2, p_out, bidx, xrows, zbuf, acc, ssem):
    c = lax.axis_index("c")
    s = lax.axis_index("s")

    def zb(i, carry):
        zbuf[i, pl.ds(0, 16)] = jnp.zeros((16,), jnp.float32)
        zbuf[i, pl.ds(16, 16)] = jnp.zeros((16,), jnp.float32)
        return carry

    lax.fori_loop(0, 128, zb, 0)
    pltpu.sync_copy(zbuf, acc.at[pl.ds(s * 128, 128)])
    plsc.subcore_barrier()
    x_c = x_pl.at[c]

    def chunk(ci, carry):
        pltpu.sync_copy(batch2.at[pl.ds(s * 25 + ci, 1)], bidx)
        pltpu.sync_copy(x_c.at[pl.ds(s * 3200 + ci * 128, 128)], xrows)
        pltpu.sync_copy(xrows, acc.at[bidx.at[0]], add=True)
        return carry

    lax.fori_loop(0, 25, chunk, 0)
    plsc.subcore_barrier()
    pltpu.sync_copy(acc.at[pl.ds(s * 128, 128)],
                    p_out.at[c].at[pl.ds(s * 128, 128)])


def _sc_pool(x_pl, batch2):
    f = pl.kernel(
        _sc_pool_body,
        out_type=jax.ShapeDtypeStruct((2, _GACC, DH), jnp.float32),
        mesh=plsc.VectorSubcoreMesh(core_axis_name="c", subcore_axis_name="s"),
        compiler_params=pltpu.CompilerParams(use_tc_tiling_on_sc=False),
        scratch_types=[
            pltpu.VMEM((1, 128), jnp.int32),
            pltpu.VMEM((128, DH), jnp.float32),
            pltpu.VMEM((128, DH), jnp.float32),
            pltpu.VMEM_SHARED((_GACC, DH), jnp.float32),
            pltpu.SemaphoreType.DMA,
        ],
    )
    return f(x_pl, batch2)


_MROWS = 1600
_MCH = 16
_MCHUNKS = E_PAD // (_MCH * 128)
_PEND = 2560


def _sc_max_body(srcp2, dstp2, c_hbm, m_out,
                 sidx, didx, pend_loc, pend_src, rows, accf, gsem):
    c = lax.axis_index("c")
    s = lax.axis_index("s")
    w = s * 2 + c
    base = w * _MROWS
    lanes = lax.broadcasted_iota(jnp.int32, (16,), 0)

    def zi(i, carry):
        accf[pl.ds(i * 16, 16)] = jnp.full((16,), NEG_BIG, jnp.float32)
        return carry

    lax.fori_loop(0, (_MROWS + 64) * D // 16, zi, 0)

    def flush(blk, off0):
        b0 = blk * 128
        pltpu.async_copy(c_hbm.at[pend_src.at[pl.ds(b0, 128)]], rows,
                         gsem).wait()
        for g in range(8):
            loc16 = pend_loc[pl.ds(b0 + g * 16, 16)]
            for l in range(16):
                loc = loc16[l]
                ab = loc * D
                for k in range(4):
                    cur = accf[pl.ds(ab + k * 16, 16)]
                    val = rows[g * 16 + l, pl.ds(k * 16, 16)]
                    accf[pl.ds(ab + k * 16, 16)] = jnp.maximum(cur, val)
        return off0

    def chunk(ci, off):
        r0 = ci * _MCH
        pltpu.sync_copy(srcp2.at[pl.ds(r0, _MCH)], sidx)
        pltpu.sync_copy(dstp2.at[pl.ds(r0, _MCH)], didx)
        for j in range(_MCH):
            for v in range(8):
                dv = didx[j, pl.ds(v * 16, 16)]
                sv = sidx[j, pl.ds(v * 16, 16)]
                loc = dv - base
                m = plsc.bitcast(loc, jnp.uint32) < jnp.uint32(_MROWS)
                key = jnp.where(m, loc, jnp.int32(0x7FFFFFF0))
                ks, vs = plsc.sort_key_val(key, sv)
                pend_loc[pl.ds(off, 16)] = ks
                pend_src[pl.ds(off, 16)] = vs
                off = off + jnp.sum(m.astype(jnp.int32))
        nblk = off >> 7
        lax.fori_loop(0, nblk, flush, off)
        rem = off & 127

        def mv(l, carry):
            pend_loc[pl.ds(l * 16, 16)] = pend_loc[pl.ds(nblk * 128 + l * 16, 16)]
            pend_src[pl.ds(l * 16, 16)] = pend_src[pl.ds(nblk * 128 + l * 16, 16)]
            return carry

        lax.fori_loop(0, 8, mv, 0)
        return rem

    off = lax.fori_loop(0, _MCHUNKS, chunk, jnp.int32(0))

    def padk(k, carry):
        pend_loc[pl.ds(off + k * 16, 16)] = jnp.full((16,), _MROWS, jnp.int32)
        pend_src[pl.ds(off + k * 16, 16)] = (lanes + 16 * k) * 32
        return carry

    lax.fori_loop(0, 8, padk, 0)
    lax.fori_loop(0, (off + 127) >> 7, flush, 0)

    pltpu.sync_copy(accf.at[pl.ds(0, _MROWS * D)],
                    m_out.at[pl.ds(base * D, _MROWS * D)])


def _sc_max(srcp2, dstp2, C):
    f = pl.kernel(
        _sc_max_body,
        out_type=jax.ShapeDtypeStruct((N_PAD * D,), jnp.float32),
        mesh=plsc.VectorSubcoreMesh(core_axis_name="c", subcore_axis_name="s"),
        compiler_params=pltpu.CompilerParams(use_tc_tiling_on_sc=False,
                                             needs_layout_passes=False),
        scratch_types=[
            pltpu.VMEM((_MCH, 128), jnp.int32),
            pltpu.VMEM((_MCH, 128), jnp.int32),
            pltpu.VMEM((_PEND,), jnp.int32),
            pltpu.VMEM((_PEND,), jnp.int32),
            pltpu.VMEM((128, D), jnp.float32),
            pltpu.VMEM(((_MROWS + 64) * D,), jnp.float32),
            pltpu.SemaphoreType.DMA,
        ],
    )
    return f(srcp2, dstp2, C)


def kernel(x, edge_attr, edge_index, batch, emb_atom_W, emb_bond_W, W_bond, b_bond,
           W_msg, b_msg, W_edge, b_edge, W_r0, b_r0, W_r1, b_r1, W_out, b_out):
    src = edge_index[0]
    dst = edge_index[1]

    W1 = W_bond[:D] - W_bond[D:]
    Wb = W_bond[D:]
    cpad = _softplus(jnp.zeros((256, D), jnp.float32))
    qc = jnp.dot(cpad, W_edge)[:1]
    Kvec = (b_msg + b_edge).reshape(1, D) + qc
    bb = b_bond.reshape(1, D)

    X, B = _embeddings(x, edge_attr[:N], emb_atom_W, emb_bond_W)

    pad_e = E_PAD - E
    spread_e = N + (jnp.arange(pad_e, dtype=jnp.int32) % 1024)
    srcp2 = jnp.concatenate([src, jnp.arange(pad_e, dtype=jnp.int32) % 1024]
                            ).reshape(E_PAD // 128, 128)
    dstp2 = jnp.concatenate([dst, spread_e]).reshape(E_PAD // 128, 128)
    spread_q = N + (jnp.arange(NQ_PAD - N, dtype=jnp.int32) % 1024)
    dsth2 = jnp.concatenate([dst[:N], spread_q]).reshape(NQ_PAD // 128, 128)

    for _ in range(2):
        A, C, P = _dense3(B, X, W1, Wb, W_msg, bb, Kvec)
        M2 = _sc_max(srcp2, dstp2, C).reshape(N_PAD, D)
        B, Qp = _postmax(A, M2, W_edge, qc)
        q_pl = _planes4(jnp.pad(Qp, ((0, NQ_PAD - N), (0, 0))))
        p_pl = _planes4(P)
        S_a = _sc_sum(0, srcp2, dstp2, p_pl, q_pl, dsth2)
        S_b = _sc_sum(1, srcp2, dstp2, p_pl, q_pl, dsth2)
        X, P = _postsum(S_a, S_b, X, Kvec, W_msg)

    out_atom = X
    out_bond = jnp.concatenate([B, _fill_tail()], axis=0)

    x_pl = _planes(jnp.pad(out_atom, ((0, N_PAD - N), (0, 0))))
    spread_b = G + (jnp.arange(N_PAD - N, dtype=jnp.int32) % 48)
    batch2 = jnp.concatenate([batch.astype(jnp.int32), spread_b]
                             ).reshape(N_PAD // 128, 128)
    pooled_pl = _sc_pool(x_pl, batch2)
    out = _readout(pooled_pl, W_r0, b_r0, W_r1, b_r1, W_out, b_out)[:, :1]
    return (out, out_atom, out_bond)
